# baseline (speedup 1.0000x reference)
import functools

import jax
import jax.numpy as jnp
import numpy as np
from jax import lax
from jax.experimental import pallas as pl
from jax.experimental.pallas import tpu as pltpu
from jax.experimental.pallas import tpu_sc as plsc

_BBOX_CLIP = float(np.log(1000.0 / 16.0))
_NBOX = 9216
_NIMG = 2
_NSUB = 16
_PER = _NBOX // _NSUB
_NV = _PER // 16
_TOPN = 1000
_NEG = np.float32(-1.0e30)
_NEGH = np.float32(-1.0e29)
_BIGI = np.int32(2**30)


def _anchors_const(H, W, stride, sizes, ratios):
    sizes = jnp.array(sizes, jnp.float32)
    ratios = jnp.array(ratios, jnp.float32)
    h_r = jnp.sqrt(ratios)
    w_r = 1.0 / h_r
    ws = (w_r[:, None] * sizes[None, :]).reshape(-1)
    hs = (h_r[:, None] * sizes[None, :]).reshape(-1)
    base = jnp.stack([-ws, -hs, ws, hs], axis=1) / 2.0
    sy, sx = jnp.meshgrid(jnp.arange(H, dtype=jnp.float32) * stride,
                          jnp.arange(W, dtype=jnp.float32) * stride, indexing='ij')
    shifts = jnp.stack([sx, sy, sx, sy], axis=-1).reshape(-1, 4)
    return (shifts[:, None, :] + base[None, :, :]).reshape(-1, 4)


def _conv_nchw(x, w, b, pad):
    y = lax.conv_general_dilated(x, w, (1, 1), [(pad, pad), (pad, pad)],
                                 dimension_numbers=('NCHW', 'OIHW', 'NCHW'))
    return y + b[None, :, None, None]


def _sc_nms_kernel(dlt_hbm, anch_hbm, probs_hbm, boxes_hbm, scores_hbm,
                   D0, D1, D2, D3, B0, B1, B2, B3, PSC, CAND, STG, OB, OS,
                   ACCK, ACCI, SHARED):
    c = lax.axis_index("c")
    s = lax.axis_index("s")
    base = s * _PER
    iota = lax.iota(jnp.int32, 16)
    fiota = iota.astype(jnp.float32)

    img_d = c * (4 * _NBOX)
    pltpu.sync_copy(dlt_hbm.at[pl.ds(img_d + 0 * _NBOX, _NBOX)], D0)
    pltpu.sync_copy(dlt_hbm.at[pl.ds(img_d + 1 * _NBOX, _NBOX)], D1)
    pltpu.sync_copy(dlt_hbm.at[pl.ds(img_d + 2 * _NBOX, _NBOX)], D2)
    pltpu.sync_copy(dlt_hbm.at[pl.ds(img_d + 3 * _NBOX, _NBOX)], D3)
    pltpu.sync_copy(anch_hbm.at[pl.ds(0 * _NBOX, _NBOX)], B0)
    pltpu.sync_copy(anch_hbm.at[pl.ds(1 * _NBOX, _NBOX)], B1)
    pltpu.sync_copy(anch_hbm.at[pl.ds(2 * _NBOX, _NBOX)], B2)
    pltpu.sync_copy(anch_hbm.at[pl.ds(3 * _NBOX, _NBOX)], B3)
    pltpu.sync_copy(probs_hbm.at[pl.ds(c * _NBOX, _NBOX)], PSC)

    def decode_body(j, _):
        sl = pl.ds(j * 16, 16)
        dx = D0[sl]
        dy = D1[sl]
        dw = D2[sl]
        dh = D3[sl]
        aw = B0[sl]
        ah = B1[sl]
        acx = B2[sl]
        acy = B3[sl]
        sc = PSC[sl]
        pcx = dx * aw + acx
        pcy = dy * ah + acy
        pw = jnp.exp(jnp.minimum(dw, _BBOX_CLIP)) * aw
        ph = jnp.exp(jnp.minimum(dh, _BBOX_CLIP)) * ah
        x1 = pcx - 0.5 * pw
        y1 = pcy - 0.5 * ph
        x2 = pcx + 0.5 * pw
        y2 = pcy + 0.5 * ph
        x1 = jnp.minimum(jnp.maximum(x1, 0.0), 512.0)
        y1 = jnp.minimum(jnp.maximum(y1, 0.0), 512.0)
        x2 = jnp.minimum(jnp.maximum(x2, 0.0), 512.0)
        y2 = jnp.minimum(jnp.maximum(y2, 0.0), 512.0)
        wd = x2 - x1
        hg = y2 - y1
        valid = (wd >= 0.01) & (hg >= 0.01)
        area = jnp.maximum(wd, 0.0) * jnp.maximum(hg, 0.0)
        gidx = (j * 16 + iota).astype(jnp.float32)
        akey = jnp.where(valid, sc, _NEG)
        skey = jnp.where(valid, _NEG, -(2.0 + gidx))
        D0[sl] = x1
        D1[sl] = y1
        D2[sl] = x2
        D3[sl] = y2
        B0[sl] = area
        B1[sl] = akey
        B2[sl] = skey
        return 0

    lax.fori_loop(0, _NBOX // 16, decode_body, 0)

    zero16 = jnp.zeros((16,), jnp.int32)
    STG[...] = jnp.full((16,), _NEG, jnp.float32)
    pltpu.sync_copy(STG.at[pl.ds(0, 8)], SHARED.at[pl.ds(0 * 128 + s * 8, 8)])
    pltpu.sync_copy(STG.at[pl.ds(0, 8)], SHARED.at[pl.ds(1 * 128 + s * 8, 8)])
    plsc.subcore_barrier()

    def round_body(rnd, carry):
        r, mode = carry
        parity = lax.rem(rnd, 2)
        in_nms = mode == 0

        in_v = jnp.full((16,), in_nms, jnp.bool_)
        ACCK[...] = jnp.full((16,), _NEG, jnp.float32)
        ACCI[...] = zero16

        def amax_body(j, _):
            sl = pl.ds(base + j * 16, 16)
            k = jnp.where(in_v, B1[sl], B2[sl])
            idx = base + j * 16 + iota
            bk = ACCK[...]
            better = k > bk
            ACCK[...] = jnp.where(better, k, bk)
            ACCI[...] = jnp.where(better, idx, ACCI[...])
            return 0

        lax.fori_loop(0, _NV, amax_body, 0, unroll=2)
        bk = ACCK[...]
        bi = ACCI[...]
        mk = jnp.max(bk)
        wi = jnp.min(jnp.where(bk == mk, bi, _BIGI))

        wi_f = plsc.bitcast(jnp.full((16,), wi, jnp.int32), jnp.float32)
        STG[...] = jnp.where(iota == 0, jnp.full((16,), mk, jnp.float32),
                             jnp.where(iota == 1, wi_f, 0.0))
        pltpu.sync_copy(STG.at[pl.ds(0, 8)], SHARED.at[pl.ds(parity * 128 + s * 8, 8)])
        plsc.subcore_barrier()
        pltpu.sync_copy(SHARED.at[pl.ds(parity * 128, 128)], CAND)
        keys = plsc.load_gather(CAND, [iota * 8])
        idxs = plsc.bitcast(plsc.load_gather(CAND, [iota * 8 + 1]), jnp.int32)
        gm = jnp.max(keys)
        gw = jnp.min(jnp.where(keys == gm, idxs, _BIGI))
        empty = gm < _NEGH
        active = jnp.logical_not(empty) & (r < _TOPN)
        gw_v = jnp.full((16,), gw, jnp.int32)

        @pl.when(in_nms & active)
        def _suppress():
            bx1 = plsc.load_gather(D0, [gw_v])
            by1 = plsc.load_gather(D1, [gw_v])
            bx2 = plsc.load_gather(D2, [gw_v])
            by2 = plsc.load_gather(D3, [gw_v])
            av = plsc.load_gather(B0, [gw_v])

            def sup_body(j, _):
                sl = pl.ds(base + j * 16, 16)
                x1 = D0[sl]
                y1 = D1[sl]
                x2 = D2[sl]
                y2 = D3[sl]
                ar = B0[sl]
                a = B1[sl]
                sv = B2[sl]
                sc = PSC[sl]
                xx1 = jnp.maximum(bx1, x1)
                yy1 = jnp.maximum(by1, y1)
                xx2 = jnp.minimum(bx2, x2)
                yy2 = jnp.minimum(by2, y2)
                inter = jnp.maximum(xx2 - xx1, 0.0) * jnp.maximum(yy2 - yy1, 0.0)
                iou = inter / (av + ar - inter + 1e-12)
                sup = iou > 0.7
                alive = a > _NEGH
                B2[sl] = jnp.where(sup & alive, sc, sv)
                B1[sl] = jnp.where(sup, _NEG, a)
                return 0

            lax.fori_loop(0, _NV, sup_body, 0, unroll=2)

        @pl.when(active)
        def _commit():
            negv = jnp.full((16,), _NEG, jnp.float32)
            lane0 = iota == 0
            plsc.store_scatter(B1, [gw_v], negv, mask=lane0)
            plsc.store_scatter(B2, [gw_v], negv, mask=lane0)

            @pl.when(s == 0)
            def _write_out():
                bx1 = plsc.load_gather(D0, [gw_v])
                by1 = plsc.load_gather(D1, [gw_v])
                bx2 = plsc.load_gather(D2, [gw_v])
                by2 = plsc.load_gather(D3, [gw_v])
                coords = jnp.where(iota == 0, bx1,
                                   jnp.where(iota == 1, by1,
                                             jnp.where(iota == 2, bx2, by2)))
                plsc.store_scatter(OB, [r * 4 + jnp.minimum(iota, 3)], coords,
                                   mask=iota < 4)
                sval = jnp.where(in_nms, gm, jnp.float32(-1.0))
                plsc.store_scatter(OS, [jnp.full((16,), r, jnp.int32)],
                                   jnp.full((16,), sval, jnp.float32), mask=lane0)

        r = r + jnp.where(active, 1, 0)
        mode = jnp.where(empty, 1, mode)
        return r, mode

    lax.fori_loop(0, _TOPN + 1, round_body, (jnp.int32(0), jnp.int32(0)))

    @pl.when(s == 0)
    def _():
        pltpu.sync_copy(OB, boxes_hbm.at[pl.ds(c * (4 * _TOPN), 4 * _TOPN)])
        pltpu.sync_copy(OS, scores_hbm.at[pl.ds(c * _TOPN, _TOPN)])


def _sc_nms(dlt_flat, anch_flat, probs_flat):
    mesh = plsc.VectorSubcoreMesh(core_axis_name="c", subcore_axis_name="s",
                                  num_cores=2, num_subcores=_NSUB)
    f32 = jnp.float32
    kern = pl.kernel(
        _sc_nms_kernel,
        out_type=(jax.ShapeDtypeStruct((_NIMG * _TOPN * 4,), f32),
                  jax.ShapeDtypeStruct((_NIMG * _TOPN,), f32)),
        mesh=mesh,
        compiler_params=pltpu.CompilerParams(needs_layout_passes=False),
        scratch_types=[
            pltpu.VMEM((_NBOX,), f32),
            pltpu.VMEM((_NBOX,), f32),
            pltpu.VMEM((_NBOX,), f32),
            pltpu.VMEM((_NBOX,), f32),
            pltpu.VMEM((_NBOX,), f32),
            pltpu.VMEM((_NBOX,), f32),
            pltpu.VMEM((_NBOX,), f32),
            pltpu.VMEM((_NBOX,), f32),
            pltpu.VMEM((_NBOX,), f32),
            pltpu.VMEM((128,), f32),
            pltpu.VMEM((16,), f32),
            pltpu.VMEM((4 * _TOPN,), f32),
            pltpu.VMEM((_TOPN,), f32),
            pltpu.VMEM((16,), f32),
            pltpu.VMEM((16,), jnp.int32),
            pltpu.VMEM_SHARED((256,), f32),
        ],
    )
    return kern(dlt_flat, anch_flat, probs_flat)


def kernel(images, features, W1, b1, Wc, bc, Wr, br):
    N, C, Hf, Wf = features.shape
    A = Wc.shape[0]
    stride = images.shape[-1] // Wf
    anchors = _anchors_const(Hf, Wf, float(stride), (64.0, 128.0, 256.0),
                             (0.5, 1.0, 2.0))
    t = jax.nn.relu(_conv_nchw(features, W1, b1, 1))
    objectness = _conv_nchw(t, Wc, bc, 0)
    pred_deltas = _conv_nchw(t, Wr, br, 0)
    obj = objectness.reshape(N, A, 1, Hf, Wf).transpose(0, 3, 4, 1, 2).reshape(N, -1)
    dlt = pred_deltas.reshape(N, A, 4, Hf, Wf).transpose(0, 3, 4, 1, 2).reshape(N, -1, 4)
    probs = jax.nn.sigmoid(lax.stop_gradient(obj))
    dlt = lax.stop_gradient(dlt)

    def nms_branch(dlt_in, probs_in):
        aw = anchors[:, 2] - anchors[:, 0]
        ah = anchors[:, 3] - anchors[:, 1]
        acx = anchors[:, 0] + 0.5 * aw
        acy = anchors[:, 1] + 0.5 * ah
        anch_flat = jnp.concatenate([aw, ah, acx, acy])
        dlt_flat = dlt_in.transpose(0, 2, 1).reshape(-1)
        probs_flat = probs_in.reshape(-1)
        bf, sf = _sc_nms(dlt_flat, anch_flat, probs_flat)
        return bf.reshape(_NIMG, _TOPN, 4), sf.reshape(_NIMG, _TOPN)

    def zero_branch(dlt_in, probs_in):
        return (jnp.zeros((_NIMG, _TOPN, 4), jnp.float32),
                jnp.zeros((_NIMG, _TOPN), jnp.float32))

    pred = lax.optimization_barrier(jnp.bool_(True))
    boxes, scores = lax.cond(pred, nms_branch, zero_branch, dlt, probs)
    return boxes, scores


# speedup vs baseline: 1.0017x; 1.0017x over previous
import functools

import jax
import jax.numpy as jnp
import numpy as np
from jax import lax
from jax.experimental import pallas as pl
from jax.experimental.pallas import tpu as pltpu
from jax.experimental.pallas import tpu_sc as plsc

_BBOX_CLIP = float(np.log(1000.0 / 16.0))
_NBOX = 9216
_NIMG = 2
_NSUB = 16
_PER = _NBOX // _NSUB
_NV = _PER // 16
_TOPN = 1000
_NEG = np.float32(-1.0e30)
_NEGH = np.float32(-1.0e29)
_BIGI = np.int32(2**30)


def _anchors_const(H, W, stride, sizes, ratios):
    sizes = jnp.array(sizes, jnp.float32)
    ratios = jnp.array(ratios, jnp.float32)
    h_r = jnp.sqrt(ratios)
    w_r = 1.0 / h_r
    ws = (w_r[:, None] * sizes[None, :]).reshape(-1)
    hs = (h_r[:, None] * sizes[None, :]).reshape(-1)
    base = jnp.stack([-ws, -hs, ws, hs], axis=1) / 2.0
    sy, sx = jnp.meshgrid(jnp.arange(H, dtype=jnp.float32) * stride,
                          jnp.arange(W, dtype=jnp.float32) * stride, indexing='ij')
    shifts = jnp.stack([sx, sy, sx, sy], axis=-1).reshape(-1, 4)
    return (shifts[:, None, :] + base[None, :, :]).reshape(-1, 4)


def _conv_nchw(x, w, b, pad):
    y = lax.conv_general_dilated(x, w, (1, 1), [(pad, pad), (pad, pad)],
                                 dimension_numbers=('NCHW', 'OIHW', 'NCHW'))
    return y + b[None, :, None, None]


def _sc_nms_kernel(dlt_hbm, anch_hbm, probs_hbm, boxes_hbm, scores_hbm,
                   D0, D1, D2, D3, B0, B1, B2, B3, PSC, CAND, STG, OB, OS,
                   ACCK, ACCI, SHARED):
    c = lax.axis_index("c")
    s = lax.axis_index("s")
    base = s * _PER
    iota = lax.iota(jnp.int32, 16)
    fiota = iota.astype(jnp.float32)

    img_d = c * (4 * _NBOX)
    pltpu.sync_copy(dlt_hbm.at[pl.ds(img_d + 0 * _NBOX, _NBOX)], D0)
    pltpu.sync_copy(dlt_hbm.at[pl.ds(img_d + 1 * _NBOX, _NBOX)], D1)
    pltpu.sync_copy(dlt_hbm.at[pl.ds(img_d + 2 * _NBOX, _NBOX)], D2)
    pltpu.sync_copy(dlt_hbm.at[pl.ds(img_d + 3 * _NBOX, _NBOX)], D3)
    pltpu.sync_copy(anch_hbm.at[pl.ds(0 * _NBOX, _NBOX)], B0)
    pltpu.sync_copy(anch_hbm.at[pl.ds(1 * _NBOX, _NBOX)], B1)
    pltpu.sync_copy(anch_hbm.at[pl.ds(2 * _NBOX, _NBOX)], B2)
    pltpu.sync_copy(anch_hbm.at[pl.ds(3 * _NBOX, _NBOX)], B3)
    pltpu.sync_copy(probs_hbm.at[pl.ds(c * _NBOX, _NBOX)], PSC)

    def decode_body(j, _):
        sl = pl.ds(j * 16, 16)
        dx = D0[sl]
        dy = D1[sl]
        dw = D2[sl]
        dh = D3[sl]
        aw = B0[sl]
        ah = B1[sl]
        acx = B2[sl]
        acy = B3[sl]
        sc = PSC[sl]
        pcx = dx * aw + acx
        pcy = dy * ah + acy
        pw = jnp.exp(jnp.minimum(dw, _BBOX_CLIP)) * aw
        ph = jnp.exp(jnp.minimum(dh, _BBOX_CLIP)) * ah
        x1 = pcx - 0.5 * pw
        y1 = pcy - 0.5 * ph
        x2 = pcx + 0.5 * pw
        y2 = pcy + 0.5 * ph
        x1 = jnp.minimum(jnp.maximum(x1, 0.0), 512.0)
        y1 = jnp.minimum(jnp.maximum(y1, 0.0), 512.0)
        x2 = jnp.minimum(jnp.maximum(x2, 0.0), 512.0)
        y2 = jnp.minimum(jnp.maximum(y2, 0.0), 512.0)
        wd = x2 - x1
        hg = y2 - y1
        valid = (wd >= 0.01) & (hg >= 0.01)
        area = jnp.maximum(wd, 0.0) * jnp.maximum(hg, 0.0)
        gidx = (j * 16 + iota).astype(jnp.float32)
        akey = jnp.where(valid, sc, _NEG)
        skey = jnp.where(valid, _NEG, -(2.0 + gidx))
        D0[sl] = x1
        D1[sl] = y1
        D2[sl] = x2
        D3[sl] = y2
        B0[sl] = area
        B1[sl] = akey
        B2[sl] = skey
        return 0

    lax.fori_loop(0, _NBOX // 16, decode_body, 0)

    zero16 = jnp.zeros((16,), jnp.int32)
    negv16 = jnp.full((16,), _NEG, jnp.float32)

    def full_amax(use_a):
        ACCK[...] = negv16
        ACCI[...] = zero16

        def amax_body(j, _):
            sl = pl.ds(base + j * 16, 16)
            k = B1[sl] if use_a else B2[sl]
            idx = base + j * 16 + iota
            bk = ACCK[...]
            better = k > bk
            ACCK[...] = jnp.where(better, k, bk)
            ACCI[...] = jnp.where(better, idx, ACCI[...])
            return 0

        lax.fori_loop(0, _NV, amax_body, 0, unroll=2)

    full_amax(True)
    STG[...] = negv16
    pltpu.sync_copy(STG.at[pl.ds(0, 8)], SHARED.at[pl.ds(0 * 128 + s * 8, 8)])
    pltpu.sync_copy(STG.at[pl.ds(0, 8)], SHARED.at[pl.ds(1 * 128 + s * 8, 8)])
    plsc.subcore_barrier()

    def round_body(rnd, carry):
        r, mode = carry
        parity = lax.rem(rnd, 2)
        in_nms = mode == 0

        bk = ACCK[...]
        bi = ACCI[...]
        mk = jnp.max(bk)
        wi = jnp.min(jnp.where(bk == mk, bi, _BIGI))
        wi_f = plsc.bitcast(jnp.full((16,), wi, jnp.int32), jnp.float32)
        STG[...] = jnp.where(iota == 0, jnp.full((16,), mk, jnp.float32),
                             jnp.where(iota == 1, wi_f, 0.0))
        pltpu.sync_copy(STG.at[pl.ds(0, 8)], SHARED.at[pl.ds(parity * 128 + s * 8, 8)])
        plsc.subcore_barrier()
        pltpu.sync_copy(SHARED.at[pl.ds(parity * 128, 128)], CAND)
        keys = plsc.load_gather(CAND, [iota * 8])
        idxs = plsc.bitcast(plsc.load_gather(CAND, [iota * 8 + 1]), jnp.int32)
        gm = jnp.max(keys)
        gw = jnp.min(jnp.where(keys == gm, idxs, _BIGI))
        empty = gm < _NEGH
        active = jnp.logical_not(empty) & (r < _TOPN)
        gw_v = jnp.full((16,), gw, jnp.int32)
        owner = (gw >= base) & (gw < base + _PER)

        @pl.when(in_nms & active)
        def _suppress():
            bx1 = plsc.load_gather(D0, [gw_v])
            by1 = plsc.load_gather(D1, [gw_v])
            bx2 = plsc.load_gather(D2, [gw_v])
            by2 = plsc.load_gather(D3, [gw_v])
            av = plsc.load_gather(B0, [gw_v])
            ACCK[...] = negv16
            ACCI[...] = zero16

            def sup_body(j, _):
                sl = pl.ds(base + j * 16, 16)
                x1 = D0[sl]
                y1 = D1[sl]
                x2 = D2[sl]
                y2 = D3[sl]
                ar = B0[sl]
                a = B1[sl]
                sv = B2[sl]
                sc = PSC[sl]
                xx1 = jnp.maximum(bx1, x1)
                yy1 = jnp.maximum(by1, y1)
                xx2 = jnp.minimum(bx2, x2)
                yy2 = jnp.minimum(by2, y2)
                inter = jnp.maximum(xx2 - xx1, 0.0) * jnp.maximum(yy2 - yy1, 0.0)
                iou = inter / (av + ar - inter + 1e-12)
                sup = iou > 0.7
                alive = a > _NEGH
                anew = jnp.where(sup, _NEG, a)
                B2[sl] = jnp.where(sup & alive, sc, sv)
                B1[sl] = anew
                idx = base + j * 16 + iota
                bk = ACCK[...]
                better = anew > bk
                ACCK[...] = jnp.where(better, anew, bk)
                ACCI[...] = jnp.where(better, idx, ACCI[...])
                return 0

            lax.fori_loop(0, _NV, sup_body, 0, unroll=2)

        @pl.when(in_nms & empty)
        def _transition():
            full_amax(False)

        @pl.when(active)
        def _commit():
            lane0 = iota == 0
            plsc.store_scatter(B1, [gw_v], negv16, mask=lane0)
            plsc.store_scatter(B2, [gw_v], negv16, mask=lane0)

            @pl.when(jnp.logical_not(in_nms) & owner)
            def _rescan():
                full_amax(False)

            @pl.when(s == 0)
            def _write_out():
                bx1 = plsc.load_gather(D0, [gw_v])
                by1 = plsc.load_gather(D1, [gw_v])
                bx2 = plsc.load_gather(D2, [gw_v])
                by2 = plsc.load_gather(D3, [gw_v])
                coords = jnp.where(iota == 0, bx1,
                                   jnp.where(iota == 1, by1,
                                             jnp.where(iota == 2, bx2, by2)))
                plsc.store_scatter(OB, [r * 4 + jnp.minimum(iota, 3)], coords,
                                   mask=iota < 4)
                sval = jnp.where(in_nms, gm, jnp.float32(-1.0))
                plsc.store_scatter(OS, [jnp.full((16,), r, jnp.int32)],
                                   jnp.full((16,), sval, jnp.float32), mask=lane0)

        r = r + jnp.where(active, 1, 0)
        mode = jnp.where(empty, 1, mode)
        return r, mode

    lax.fori_loop(0, _TOPN + 1, round_body, (jnp.int32(0), jnp.int32(0)))

    @pl.when(s == 0)
    def _():
        pltpu.sync_copy(OB, boxes_hbm.at[pl.ds(c * (4 * _TOPN), 4 * _TOPN)])
        pltpu.sync_copy(OS, scores_hbm.at[pl.ds(c * _TOPN, _TOPN)])


def _sc_nms(dlt_flat, anch_flat, probs_flat):
    mesh = plsc.VectorSubcoreMesh(core_axis_name="c", subcore_axis_name="s",
                                  num_cores=2, num_subcores=_NSUB)
    f32 = jnp.float32
    kern = pl.kernel(
        _sc_nms_kernel,
        out_type=(jax.ShapeDtypeStruct((_NIMG * _TOPN * 4,), f32),
                  jax.ShapeDtypeStruct((_NIMG * _TOPN,), f32)),
        mesh=mesh,
        compiler_params=pltpu.CompilerParams(needs_layout_passes=False),
        scratch_types=[
            pltpu.VMEM((_NBOX,), f32),
            pltpu.VMEM((_NBOX,), f32),
            pltpu.VMEM((_NBOX,), f32),
            pltpu.VMEM((_NBOX,), f32),
            pltpu.VMEM((_NBOX,), f32),
            pltpu.VMEM((_NBOX,), f32),
            pltpu.VMEM((_NBOX,), f32),
            pltpu.VMEM((_NBOX,), f32),
            pltpu.VMEM((_NBOX,), f32),
            pltpu.VMEM((128,), f32),
            pltpu.VMEM((16,), f32),
            pltpu.VMEM((4 * _TOPN,), f32),
            pltpu.VMEM((_TOPN,), f32),
            pltpu.VMEM((16,), f32),
            pltpu.VMEM((16,), jnp.int32),
            pltpu.VMEM_SHARED((256,), f32),
        ],
    )
    return kern(dlt_flat, anch_flat, probs_flat)


def kernel(images, features, W1, b1, Wc, bc, Wr, br):
    N, C, Hf, Wf = features.shape
    A = Wc.shape[0]
    stride = images.shape[-1] // Wf
    anchors = _anchors_const(Hf, Wf, float(stride), (64.0, 128.0, 256.0),
                             (0.5, 1.0, 2.0))
    t = jax.nn.relu(_conv_nchw(features, W1, b1, 1))
    objectness = _conv_nchw(t, Wc, bc, 0)
    pred_deltas = _conv_nchw(t, Wr, br, 0)
    obj = objectness.reshape(N, A, 1, Hf, Wf).transpose(0, 3, 4, 1, 2).reshape(N, -1)
    dlt = pred_deltas.reshape(N, A, 4, Hf, Wf).transpose(0, 3, 4, 1, 2).reshape(N, -1, 4)
    probs = jax.nn.sigmoid(lax.stop_gradient(obj))
    dlt = lax.stop_gradient(dlt)

    def nms_branch(dlt_in, probs_in):
        aw = anchors[:, 2] - anchors[:, 0]
        ah = anchors[:, 3] - anchors[:, 1]
        acx = anchors[:, 0] + 0.5 * aw
        acy = anchors[:, 1] + 0.5 * ah
        anch_flat = jnp.concatenate([aw, ah, acx, acy])
        dlt_flat = dlt_in.transpose(0, 2, 1).reshape(-1)
        probs_flat = probs_in.reshape(-1)
        bf, sf = _sc_nms(dlt_flat, anch_flat, probs_flat)
        return bf.reshape(_NIMG, _TOPN, 4), sf.reshape(_NIMG, _TOPN)

    def zero_branch(dlt_in, probs_in):
        return (jnp.zeros((_NIMG, _TOPN, 4), jnp.float32),
                jnp.zeros((_NIMG, _TOPN), jnp.float32))

    pred = lax.optimization_barrier(jnp.bool_(True))
    boxes, scores = lax.cond(pred, nms_branch, zero_branch, dlt, probs)
    return boxes, scores
